# baseline (speedup 1.0000x reference)
import functools

import jax
import jax.numpy as jnp
from jax import lax
from jax.experimental import pallas as pl
from jax.experimental.pallas import tpu as pltpu
from jax.experimental.pallas import tpu_sc as plsc

N_NODES = 10000
D = 128
DH = D // 2
E = 320000

NC = 2
NS = 16
NW = NC * NS
NPAD = 10240
G = 80
EPT = E // NW
NG = EPT // G
NGH = E // NS // G
RPT = NPAD // NS
ZR = 80
PB = 40

_MESH = plsc.VectorSubcoreMesh(core_axis_name="c", subcore_axis_name="s")
_SC_PARAMS = pltpu.CompilerParams(needs_layout_passes=False,
                                  use_tc_tiling_on_sc=False)


def _rsqrt16(x):
  i = plsc.bitcast(x, jnp.int32)
  i = jnp.int32(0x5F3759DF) - lax.shift_right_logical(i, 1)
  y = plsc.bitcast(i, jnp.float32)
  for _ in range(3):
    y = y * (1.5 - 0.5 * x * y * y)
  return y


def _zero_ref16(ref, n16):
  zero16 = jnp.zeros((16,), jnp.float32)

  def zh(i, carry):
    ref[pl.ds(i * 16, 16)] = zero16
    return carry

  lax.fori_loop(0, n16, zh, 0)


def _prop_pipeline(table, srcv, dstv, rows0, rows1, acc, sem0, sem1,
                   scalar_work):
  def wait_gather(rows, sem):
    pltpu.make_async_copy(table.at[pl.ds(0, G)], rows, sem).wait()

  pltpu.async_copy(table.at[srcv.at[0]], rows0, sem0)

  def pair_body(gg, carry):
    g0 = 2 * gg
    pltpu.async_copy(table.at[srcv.at[g0 + 1]], rows1, sem1)
    scalar_work(g0)
    wait_gather(rows0, sem0)
    pltpu.sync_copy(rows0, acc.at[dstv.at[g0]], add=True)
    pltpu.async_copy(table.at[srcv.at[g0 + 2]], rows0, sem0)
    scalar_work(g0 + 1)
    wait_gather(rows1, sem1)
    pltpu.sync_copy(rows1, acc.at[dstv.at[g0 + 1]], add=True)
    return carry

  lax.fori_loop(0, (NG - 1) // 2, pair_body, 0)
  scalar_work(NG - 1)
  wait_gather(rows0, sem0)
  pltpu.sync_copy(rows0, acc.at[dstv.at[NG - 1]], add=True)


@functools.partial(
    pl.kernel,
    out_type=jax.ShapeDtypeStruct((NC, NPAD), jnp.float32),
    mesh=_MESH,
    compiler_params=_SC_PARAMS,
    scratch_types=[
        pltpu.VMEM((NC, NG, G), jnp.int32),
        pltpu.VMEM((NPAD,), jnp.float32),
        pltpu.VMEM((NS, RPT), jnp.float32),
        pltpu.VMEM_SHARED((NS, NPAD), jnp.float32),
    ],
)
def _pass1a(dst_h, dinv_h, dstH_v, hist, rbuf, shared):
  cid = lax.axis_index("c")
  sid = lax.axis_index("s")
  ones16 = jnp.full((16,), 1.0, jnp.float32)

  _zero_ref16(hist, NPAD // 16)
  pltpu.sync_copy(dst_h.at[pl.ds(NC * sid, NC)], dstH_v)

  def h_body(g, carry):
    for a in range(NC):
      for j in range(G // 16):
        idx = dstH_v[a, g, pl.ds(j * 16, 16)]
        plsc.addupdate_scatter(hist, [idx], ones16)
    return carry

  lax.fori_loop(0, NG, h_body, 0)

  pltpu.sync_copy(hist, shared.at[sid])
  plsc.subcore_barrier()
  pltpu.sync_copy(shared.at[:, pl.ds(sid * RPT, RPT)], rbuf)

  def r_body(cv, carry):
    a = rbuf[0, pl.ds(cv * 16, 16)]
    for r in range(1, NS):
      a = a + rbuf[r, pl.ds(cv * 16, 16)]
    x = a + 1.0
    hist[pl.ds(cv * 16, 16)] = _rsqrt16(x)
    return carry

  lax.fori_loop(0, RPT // 16, r_body, 0)
  pltpu.sync_copy(hist.at[pl.ds(0, RPT)],
                  dinv_h.at[cid, pl.ds(sid * RPT, RPT)])


@functools.partial(
    pl.kernel,
    out_type=[
        jax.ShapeDtypeStruct((NC, NPAD, 2, DH), jnp.float32),
        jax.ShapeDtypeStruct((NW, NPAD), jnp.float32),
        jax.ShapeDtypeStruct((NC, 2 * NPAD, DH), jnp.float32),
    ],
    mesh=_MESH,
    compiler_params=_SC_PARAMS,
    scratch_types=[
        pltpu.VMEM((NG, G), jnp.int32),
        pltpu.VMEM((NG, G), jnp.int32),
        pltpu.VMEM((G, DH), jnp.float32),
        pltpu.VMEM((G, DH), jnp.float32),
        pltpu.VMEM((ZR, DH), jnp.float32),
        pltpu.VMEM((ZR, DH), jnp.float32),
        pltpu.VMEM((NPAD,), jnp.float32),
        pltpu.VMEM((NPAD,), jnp.float32),
        pltpu.VMEM_SHARED((NPAD, DH), jnp.float32),
        pltpu.SemaphoreType.DMA,
        pltpu.SemaphoreType.DMA,
    ],
)
def _pass1b(srcs_h, dst_h, emb_h, dinv_h, parts_h, sparts_h, p0t_h,
            srcv, dstv, rows0, rows1, zbuf, embbuf, dvecv, hist,
            acc, sem0, sem1):
  cid = lax.axis_index("c")
  sid = lax.axis_index("s")
  wid = cid * NS + sid
  zero16 = jnp.zeros((16,), jnp.float32)

  def zb(i, carry):
    for j in range(DH // 16):
      zbuf[i, pl.ds(j * 16, 16)] = zero16
    return carry

  lax.fori_loop(0, ZR, zb, 0)
  pltpu.sync_copy(dinv_h.at[cid], dvecv)

  base0 = 2 * sid * RPT
  for k in range(2 * RPT // ZR):
    base = base0 + k * ZR
    pltpu.sync_copy(emb_h.at[pl.ds(base, ZR)], embbuf)

    def prow(r, carry):
      t16 = jnp.full((16,), base + r, jnp.int32)
      node16 = lax.shift_right_logical(t16, 1)
      dv = plsc.load_gather(dvecv, [node16])
      for c in range(DH // 16):
        embbuf[r, pl.ds(c * 16, 16)] = embbuf[r, pl.ds(c * 16, 16)] * dv
      return carry

    lax.fori_loop(0, ZR, prow, 0)
    pltpu.sync_copy(embbuf, p0t_h.at[cid, pl.ds(base, ZR)])

  _zero_ref16(hist, NPAD // 16)
  for k in range(RPT // ZR):
    pltpu.sync_copy(zbuf, acc.at[pl.ds(sid * RPT + k * ZR, ZR)])
  pltpu.sync_copy(dst_h.at[wid], dstv)
  plsc.subcore_barrier()

  for h in range(2):
    pltpu.sync_copy(srcs_h.at[h, wid], srcv)
    if h == 1:
      plsc.subcore_barrier()

    if h == 0:
      def scalar_work(g):
        for j in range(G // 16):
          sidx = lax.shift_right_logical(srcv[g, pl.ds(j * 16, 16)], 1)
          didx = dstv[g, pl.ds(j * 16, 16)]
          vals = plsc.load_gather(dvecv, [sidx])
          plsc.addupdate_scatter(hist, [didx], vals)
    else:
      def scalar_work(g):
        del g

    _prop_pipeline(p0t_h.at[cid], srcv, dstv, rows0, rows1, acc, sem0, sem1,
                   scalar_work)
    plsc.subcore_barrier()
    pltpu.sync_copy(acc.at[pl.ds(sid * RPT, RPT)],
                    parts_h.at[cid, pl.ds(sid * RPT, RPT), h])
    if h == 0:
      for k in range(RPT // ZR):
        pltpu.sync_copy(zbuf, acc.at[pl.ds(sid * RPT + k * ZR, ZR)])
  pltpu.sync_copy(hist, sparts_h.at[wid])


@functools.partial(
    pl.kernel,
    out_type=[
        jax.ShapeDtypeStruct((NC, NPAD, 2, DH), jnp.float32),
        jax.ShapeDtypeStruct((NC, 2 * NPAD, DH), jnp.float32),
    ],
    mesh=_MESH,
    compiler_params=_SC_PARAMS,
    scratch_types=[
        pltpu.VMEM((NG, G), jnp.int32),
        pltpu.VMEM((NG, G), jnp.int32),
        pltpu.VMEM((G, DH), jnp.float32),
        pltpu.VMEM((G, DH), jnp.float32),
        pltpu.VMEM((ZR, DH), jnp.float32),
        pltpu.VMEM((RPT,), jnp.float32),
        pltpu.VMEM((PB, 2, DH), jnp.float32),
        pltpu.VMEM((PB, 2, DH), jnp.float32),
        pltpu.VMEM((2 * PB, DH), jnp.float32),
        pltpu.VMEM_SHARED((NPAD, DH), jnp.float32),
        pltpu.SemaphoreType.DMA,
        pltpu.SemaphoreType.DMA,
    ],
)
def _pass2(srcs_h, dst_h, dinv_h, parts0_h, p0t_h, parts_h, p1t_h,
           srcv, dstv, rows0, rows1, zbuf, dloc, bufA, bufB, pbuf, acc,
           sem0, sem1):
  cid = lax.axis_index("c")
  sid = lax.axis_index("s")
  wid = cid * NS + sid
  zero16 = jnp.zeros((16,), jnp.float32)

  def zb(i, carry):
    for j in range(DH // 16):
      zbuf[i, pl.ds(j * 16, 16)] = zero16
    return carry

  lax.fori_loop(0, ZR, zb, 0)

  pltpu.sync_copy(dinv_h.at[cid, pl.ds(sid * RPT, RPT)], dloc)

  for k in range(RPT // PB):
    nb = sid * RPT + k * PB
    lb = k * PB
    pltpu.sync_copy(parts0_h.at[0, pl.ds(nb, PB)], bufA)
    pltpu.sync_copy(parts0_h.at[1, pl.ds(nb, PB)], bufB)
    pltpu.sync_copy(p0t_h.at[cid, pl.ds(2 * nb, 2 * PB)], pbuf)

    def prow(r, carry):
      l16 = jnp.full((16,), lb + r, jnp.int32)
      dv = plsc.load_gather(dloc, [l16])
      dv2 = dv * dv
      for hh in range(2):
        for c in range(DH // 16):
          sl = pl.ds(c * 16, 16)
          pbuf[2 * r + hh, sl] = (
              bufA[r, hh, sl] + bufB[r, hh, sl] + pbuf[2 * r + hh, sl]) * dv2
      return carry

    lax.fori_loop(0, PB, prow, 0)
    pltpu.sync_copy(pbuf, p1t_h.at[cid, pl.ds(2 * nb, 2 * PB)])

  for k in range(RPT // ZR):
    pltpu.sync_copy(zbuf, acc.at[pl.ds(sid * RPT + k * ZR, ZR)])
  pltpu.sync_copy(dst_h.at[wid], dstv)
  plsc.subcore_barrier()

  def no_scalar(g):
    del g

  for h in range(2):
    pltpu.sync_copy(srcs_h.at[h, wid], srcv)
    if h == 1:
      plsc.subcore_barrier()
    _prop_pipeline(p1t_h.at[cid], srcv, dstv, rows0, rows1, acc, sem0, sem1,
                   no_scalar)
    plsc.subcore_barrier()
    pltpu.sync_copy(acc.at[pl.ds(sid * RPT, RPT)],
                    parts_h.at[cid, pl.ds(sid * RPT, RPT), h])
    if h == 0:
      for k in range(RPT // ZR):
        pltpu.sync_copy(zbuf, acc.at[pl.ds(sid * RPT + k * ZR, ZR)])


_RB = 1024


def _final_body(parts_ref, p1_ref, dinv_ref, sparts_ref,
                w1_ref, b1_ref, w2_ref, b2_ref, out_ref):
  d = dinv_ref[0]
  p1r = p1_ref[0].reshape(_RB, 2, DH)
  q2 = (parts_ref[0] + parts_ref[1] + p1r) * d[:, None, None]
  w12 = jnp.dot(w1_ref[...], w2_ref[...], preferred_element_type=jnp.float32)
  b12 = jnp.dot(b1_ref[...][None, :], w2_ref[...],
                preferred_element_type=jnp.float32)[0]
  s = d * (jnp.sum(sparts_ref[...], axis=0) + d)
  out_ref[...] = (
      jnp.dot(q2[:, 0, :], w12[:DH, :], preferred_element_type=jnp.float32)
      + jnp.dot(q2[:, 1, :], w12[DH:, :], preferred_element_type=jnp.float32)
      + s[:, None] * b12[None, :] + b2_ref[...][None, :])


def _final_call(parts1, p1t, dinv, sparts, W1, b1, W2, b2):
  return pl.pallas_call(
      _final_body,
      grid=(NPAD // _RB,),
      in_specs=[
          pl.BlockSpec((NC, _RB, 2, DH), lambda i: (0, i, 0, 0)),
          pl.BlockSpec((1, 2 * _RB, DH), lambda i: (0, i, 0)),
          pl.BlockSpec((NC, _RB), lambda i: (0, i)),
          pl.BlockSpec((NW, _RB), lambda i: (0, i)),
          pl.BlockSpec((D, 2 * D), lambda i: (0, 0)),
          pl.BlockSpec((2 * D,), lambda i: (0,)),
          pl.BlockSpec((2 * D, D), lambda i: (0, 0)),
          pl.BlockSpec((D,), lambda i: (0,)),
      ],
      out_specs=pl.BlockSpec((_RB, D), lambda i: (i, 0)),
      out_shape=jax.ShapeDtypeStruct((NPAD, D), jnp.float32),
  )(parts1, p1t, dinv, sparts, W1, b1, W2, b2)


def kernel(edge_index, edge_weight, emb, W1, b1, W2, b2):
  src = edge_index[0].astype(jnp.int32)
  dst = edge_index[1].astype(jnp.int32)
  src2 = src * 2
  srcs = jnp.stack([src2, src2 + 1]).reshape(2, NW, NG, G)
  dst3 = dst.reshape(NW, NG, G)
  emb2 = jnp.pad(emb, ((0, NPAD - N_NODES), (0, 0))).reshape(2 * NPAD, DH)

  dinv = _pass1a(dst3)
  parts0, sparts, p0t = _pass1b(srcs, dst3, emb2, dinv)
  parts1, p1t = _pass2(srcs, dst3, dinv, parts0, p0t)
  out = _final_call(parts1, p1t, dinv, sparts, W1, b1, W2, b2)
  return out[:N_NODES]


# speedup vs baseline: 1.0890x; 1.0890x over previous
import functools

import jax
import jax.numpy as jnp
from jax import lax
from jax.experimental import pallas as pl
from jax.experimental.pallas import tpu as pltpu
from jax.experimental.pallas import tpu_sc as plsc

N_NODES = 10000
D = 128
DH = D // 2
E = 320000

NC = 2
NS = 16
NW = NC * NS
NPAD = 10240
G = 80
EPT = E // NW
NG = EPT // G
NGH = E // NS // G
RPT = NPAD // NS
ZR = 80
PZ = 160
PB = 80

_MESH = plsc.VectorSubcoreMesh(core_axis_name="c", subcore_axis_name="s")
_SC_PARAMS = pltpu.CompilerParams(needs_layout_passes=False,
                                  use_tc_tiling_on_sc=False)


def _rsqrt16(x):
  i = plsc.bitcast(x, jnp.int32)
  i = jnp.int32(0x5F3759DF) - lax.shift_right_logical(i, 1)
  y = plsc.bitcast(i, jnp.float32)
  for _ in range(3):
    y = y * (1.5 - 0.5 * x * y * y)
  return y


def _zero_ref16(ref, n16):
  zero16 = jnp.zeros((16,), jnp.float32)

  def zh(i, carry):
    ref[pl.ds(i * 16, 16)] = zero16
    return carry

  lax.fori_loop(0, n16, zh, 0)


def _prop_pipeline(table, srcv, dstv, rows0, rows1, acc, sem0, sem1,
                   scalar_work):
  def wait_gather(rows, sem):
    pltpu.make_async_copy(table.at[pl.ds(0, G)], rows, sem).wait()

  pltpu.async_copy(table.at[srcv.at[0]], rows0, sem0)

  def pair_body(gg, carry):
    g0 = 2 * gg
    pltpu.async_copy(table.at[srcv.at[g0 + 1]], rows1, sem1)
    scalar_work(g0)
    wait_gather(rows0, sem0)
    pltpu.sync_copy(rows0, acc.at[dstv.at[g0]], add=True)
    pltpu.async_copy(table.at[srcv.at[g0 + 2]], rows0, sem0)
    scalar_work(g0 + 1)
    wait_gather(rows1, sem1)
    pltpu.sync_copy(rows1, acc.at[dstv.at[g0 + 1]], add=True)
    return carry

  lax.fori_loop(0, (NG - 1) // 2, pair_body, 0)
  scalar_work(NG - 1)
  wait_gather(rows0, sem0)
  pltpu.sync_copy(rows0, acc.at[dstv.at[NG - 1]], add=True)


@functools.partial(
    pl.kernel,
    out_type=jax.ShapeDtypeStruct((NC, NPAD), jnp.float32),
    mesh=_MESH,
    compiler_params=_SC_PARAMS,
    scratch_types=[
        pltpu.VMEM((NC, NG, G), jnp.int32),
        pltpu.VMEM((NPAD,), jnp.float32),
        pltpu.VMEM((NS, RPT), jnp.float32),
        pltpu.VMEM_SHARED((NS, NPAD), jnp.float32),
    ],
)
def _pass1a(dst_h, dinv_h, dstH_v, hist, rbuf, shared):
  cid = lax.axis_index("c")
  sid = lax.axis_index("s")
  ones16 = jnp.full((16,), 1.0, jnp.float32)

  _zero_ref16(hist, NPAD // 16)
  pltpu.sync_copy(dst_h.at[pl.ds(NC * sid, NC)], dstH_v)

  def h_body(g, carry):
    for a in range(NC):
      for j in range(G // 16):
        idx = dstH_v[a, g, pl.ds(j * 16, 16)]
        plsc.addupdate_scatter(hist, [idx], ones16)
    return carry

  lax.fori_loop(0, NG, h_body, 0)

  pltpu.sync_copy(hist, shared.at[sid])
  plsc.subcore_barrier()
  pltpu.sync_copy(shared.at[:, pl.ds(sid * RPT, RPT)], rbuf)

  def r_body(cv, carry):
    a = rbuf[0, pl.ds(cv * 16, 16)]
    for r in range(1, NS):
      a = a + rbuf[r, pl.ds(cv * 16, 16)]
    x = a + 1.0
    hist[pl.ds(cv * 16, 16)] = _rsqrt16(x)
    return carry

  lax.fori_loop(0, RPT // 16, r_body, 0)
  pltpu.sync_copy(hist.at[pl.ds(0, RPT)],
                  dinv_h.at[cid, pl.ds(sid * RPT, RPT)])


@functools.partial(
    pl.kernel,
    out_type=[
        jax.ShapeDtypeStruct((NC, NPAD, 2, DH), jnp.float32),
        jax.ShapeDtypeStruct((NW, NPAD), jnp.float32),
        jax.ShapeDtypeStruct((NC, 2 * NPAD, DH), jnp.float32),
    ],
    mesh=_MESH,
    compiler_params=_SC_PARAMS,
    scratch_types=[
        pltpu.VMEM((NG, G), jnp.int32),
        pltpu.VMEM((NG, G), jnp.int32),
        pltpu.VMEM((G, DH), jnp.float32),
        pltpu.VMEM((G, DH), jnp.float32),
        pltpu.VMEM((ZR, DH), jnp.float32),
        pltpu.VMEM((PZ, DH), jnp.float32),
        pltpu.VMEM((PZ, DH), jnp.float32),
        pltpu.VMEM((NPAD,), jnp.float32),
        pltpu.VMEM((NPAD,), jnp.float32),
        pltpu.VMEM_SHARED((NPAD, DH), jnp.float32),
        pltpu.SemaphoreType.DMA,
        pltpu.SemaphoreType.DMA,
    ],
)
def _pass1b(srcs_h, dst_h, emb_h, dinv_h, parts_h, sparts_h, p0t_h,
            srcv, dstv, rows0, rows1, zbuf, embbuf0, embbuf1, dvecv, hist,
            acc, sem0, sem1):
  cid = lax.axis_index("c")
  sid = lax.axis_index("s")
  wid = cid * NS + sid
  zero16 = jnp.zeros((16,), jnp.float32)

  def zb(i, carry):
    for j in range(DH // 16):
      zbuf[i, pl.ds(j * 16, 16)] = zero16
    return carry

  lax.fori_loop(0, ZR, zb, 0)
  pltpu.sync_copy(dinv_h.at[cid], dvecv)

  base0 = 2 * sid * RPT
  nkc = 2 * RPT // PZ
  bufs = (embbuf0, embbuf1)
  sems = (sem0, sem1)
  pltpu.async_copy(emb_h.at[pl.ds(base0, PZ)], embbuf0, sem0)
  for k in range(nkc):
    b = k % 2
    buf, sem = bufs[b], sems[b]
    base = base0 + k * PZ
    if k + 1 < nkc:
      pltpu.async_copy(emb_h.at[pl.ds(base + PZ, PZ)], bufs[1 - b],
                       sems[1 - b])
    pltpu.make_async_copy(emb_h.at[pl.ds(0, PZ)], buf, sem).wait()

    def prow(n, carry, buf=buf, base=base):
      node16 = jnp.full((16,), base // 2 + n, jnp.int32)
      dv = plsc.load_gather(dvecv, [node16])
      for rr in range(2):
        for c in range(DH // 16):
          sl = pl.ds(c * 16, 16)
          buf[2 * n + rr, sl] = buf[2 * n + rr, sl] * dv
      return carry

    lax.fori_loop(0, PZ // 2, prow, 0)
    pltpu.sync_copy(buf, p0t_h.at[cid, pl.ds(base, PZ)])

  _zero_ref16(hist, NPAD // 16)
  for k in range(RPT // ZR):
    pltpu.sync_copy(zbuf, acc.at[pl.ds(sid * RPT + k * ZR, ZR)])
  pltpu.sync_copy(dst_h.at[wid], dstv)
  plsc.subcore_barrier()

  for h in range(2):
    pltpu.sync_copy(srcs_h.at[h, wid], srcv)
    if h == 1:
      plsc.subcore_barrier()

    if h == 0:
      def scalar_work(g):
        for j in range(G // 16):
          sidx = lax.shift_right_logical(srcv[g, pl.ds(j * 16, 16)], 1)
          didx = dstv[g, pl.ds(j * 16, 16)]
          vals = plsc.load_gather(dvecv, [sidx])
          plsc.addupdate_scatter(hist, [didx], vals)
    else:
      def scalar_work(g):
        del g

    _prop_pipeline(p0t_h.at[cid], srcv, dstv, rows0, rows1, acc, sem0, sem1,
                   scalar_work)
    plsc.subcore_barrier()
    pltpu.sync_copy(acc.at[pl.ds(sid * RPT, RPT)],
                    parts_h.at[cid, pl.ds(sid * RPT, RPT), h])
    if h == 0:
      for k in range(RPT // ZR):
        pltpu.sync_copy(zbuf, acc.at[pl.ds(sid * RPT + k * ZR, ZR)])
  pltpu.sync_copy(hist, sparts_h.at[wid])


@functools.partial(
    pl.kernel,
    out_type=[
        jax.ShapeDtypeStruct((NC, NPAD, 2, DH), jnp.float32),
        jax.ShapeDtypeStruct((NC, 2 * NPAD, DH), jnp.float32),
    ],
    mesh=_MESH,
    compiler_params=_SC_PARAMS,
    scratch_types=[
        pltpu.VMEM((NG, G), jnp.int32),
        pltpu.VMEM((NG, G), jnp.int32),
        pltpu.VMEM((G, DH), jnp.float32),
        pltpu.VMEM((G, DH), jnp.float32),
        pltpu.VMEM((ZR, DH), jnp.float32),
        pltpu.VMEM((RPT,), jnp.float32),
        pltpu.VMEM((PB, 2, DH), jnp.float32),
        pltpu.VMEM((PB, 2, DH), jnp.float32),
        pltpu.VMEM((2 * PB, DH), jnp.float32),
        pltpu.VMEM_SHARED((NPAD, DH), jnp.float32),
        pltpu.SemaphoreType.DMA,
        pltpu.SemaphoreType.DMA,
    ],
)
def _pass2(srcs_h, dst_h, dinv_h, parts0_h, p0t_h, parts_h, p1t_h,
           srcv, dstv, rows0, rows1, zbuf, dloc, bufA, bufB, pbuf, acc,
           sem0, sem1):
  cid = lax.axis_index("c")
  sid = lax.axis_index("s")
  wid = cid * NS + sid
  zero16 = jnp.zeros((16,), jnp.float32)

  def zb(i, carry):
    for j in range(DH // 16):
      zbuf[i, pl.ds(j * 16, 16)] = zero16
    return carry

  lax.fori_loop(0, ZR, zb, 0)

  pltpu.sync_copy(dinv_h.at[cid, pl.ds(sid * RPT, RPT)], dloc)

  for k in range(RPT // PB):
    nb = sid * RPT + k * PB
    lb = k * PB
    da = pltpu.async_copy(parts0_h.at[0, pl.ds(nb, PB)], bufA, sem0)
    db = pltpu.async_copy(parts0_h.at[1, pl.ds(nb, PB)], bufB, sem1)
    pltpu.sync_copy(p0t_h.at[cid, pl.ds(2 * nb, 2 * PB)], pbuf)
    da.wait()
    db.wait()

    def prow(r, carry):
      l16 = jnp.full((16,), lb + r, jnp.int32)
      dv = plsc.load_gather(dloc, [l16])
      dv2 = dv * dv
      for hh in range(2):
        for c in range(DH // 16):
          sl = pl.ds(c * 16, 16)
          pbuf[2 * r + hh, sl] = (
              bufA[r, hh, sl] + bufB[r, hh, sl] + pbuf[2 * r + hh, sl]) * dv2
      return carry

    lax.fori_loop(0, PB, prow, 0)
    pltpu.sync_copy(pbuf, p1t_h.at[cid, pl.ds(2 * nb, 2 * PB)])

  for k in range(RPT // ZR):
    pltpu.sync_copy(zbuf, acc.at[pl.ds(sid * RPT + k * ZR, ZR)])
  pltpu.sync_copy(dst_h.at[wid], dstv)
  plsc.subcore_barrier()

  def no_scalar(g):
    del g

  for h in range(2):
    pltpu.sync_copy(srcs_h.at[h, wid], srcv)
    if h == 1:
      plsc.subcore_barrier()
    _prop_pipeline(p1t_h.at[cid], srcv, dstv, rows0, rows1, acc, sem0, sem1,
                   no_scalar)
    plsc.subcore_barrier()
    pltpu.sync_copy(acc.at[pl.ds(sid * RPT, RPT)],
                    parts_h.at[cid, pl.ds(sid * RPT, RPT), h])
    if h == 0:
      for k in range(RPT // ZR):
        pltpu.sync_copy(zbuf, acc.at[pl.ds(sid * RPT + k * ZR, ZR)])


_RB = 1024


def _final_body(parts_ref, p1_ref, dinv_ref, sparts_ref,
                w1_ref, b1_ref, w2_ref, b2_ref, out_ref):
  d = dinv_ref[0]
  p1r = p1_ref[0].reshape(_RB, 2, DH)
  q2 = (parts_ref[0] + parts_ref[1] + p1r) * d[:, None, None]
  w12 = jnp.dot(w1_ref[...], w2_ref[...], preferred_element_type=jnp.float32)
  b12 = jnp.dot(b1_ref[...][None, :], w2_ref[...],
                preferred_element_type=jnp.float32)[0]
  s = d * (jnp.sum(sparts_ref[...], axis=0) + d)
  out_ref[...] = (
      jnp.dot(q2[:, 0, :], w12[:DH, :], preferred_element_type=jnp.float32)
      + jnp.dot(q2[:, 1, :], w12[DH:, :], preferred_element_type=jnp.float32)
      + s[:, None] * b12[None, :] + b2_ref[...][None, :])


def _final_call(parts1, p1t, dinv, sparts, W1, b1, W2, b2):
  return pl.pallas_call(
      _final_body,
      grid=(NPAD // _RB,),
      in_specs=[
          pl.BlockSpec((NC, _RB, 2, DH), lambda i: (0, i, 0, 0)),
          pl.BlockSpec((1, 2 * _RB, DH), lambda i: (0, i, 0)),
          pl.BlockSpec((NC, _RB), lambda i: (0, i)),
          pl.BlockSpec((NW, _RB), lambda i: (0, i)),
          pl.BlockSpec((D, 2 * D), lambda i: (0, 0)),
          pl.BlockSpec((2 * D,), lambda i: (0,)),
          pl.BlockSpec((2 * D, D), lambda i: (0, 0)),
          pl.BlockSpec((D,), lambda i: (0,)),
      ],
      out_specs=pl.BlockSpec((_RB, D), lambda i: (i, 0)),
      out_shape=jax.ShapeDtypeStruct((NPAD, D), jnp.float32),
  )(parts1, p1t, dinv, sparts, W1, b1, W2, b2)


def kernel(edge_index, edge_weight, emb, W1, b1, W2, b2):
  src = edge_index[0].astype(jnp.int32)
  dst = edge_index[1].astype(jnp.int32)
  src2 = src * 2
  srcs = jnp.stack([src2, src2 + 1]).reshape(2, NW, NG, G)
  dst3 = dst.reshape(NW, NG, G)
  emb2 = jnp.pad(emb, ((0, NPAD - N_NODES), (0, 0))).reshape(2 * NPAD, DH)

  dinv = _pass1a(dst3)
  parts0, sparts, p0t = _pass1b(srcs, dst3, emb2, dinv)
  parts1, p1t = _pass2(srcs, dst3, dinv, parts0, p0t)
  out = _final_call(parts1, p1t, dinv, sparts, W1, b1, W2, b2)
  return out[:N_NODES]


# speedup vs baseline: 1.0987x; 1.0089x over previous
import functools

import jax
import jax.numpy as jnp
from jax import lax
from jax.experimental import pallas as pl
from jax.experimental.pallas import tpu as pltpu
from jax.experimental.pallas import tpu_sc as plsc

N_NODES = 10000
D = 128
DH = D // 2
E = 320000

NC = 2
NS = 16
NW = NC * NS
NPAD = 10240
G = 80
EPT = E // NW
NG = EPT // G
NGH = E // NS // G
RPT = NPAD // NS
ZR = 80
PZ = 80
PB = 40

_MESH = plsc.VectorSubcoreMesh(core_axis_name="c", subcore_axis_name="s")
_SC_PARAMS = pltpu.CompilerParams(needs_layout_passes=False,
                                  use_tc_tiling_on_sc=False)


def _rsqrt16(x):
  i = plsc.bitcast(x, jnp.int32)
  i = jnp.int32(0x5F3759DF) - lax.shift_right_logical(i, 1)
  y = plsc.bitcast(i, jnp.float32)
  for _ in range(3):
    y = y * (1.5 - 0.5 * x * y * y)
  return y


def _zero_ref16(ref, n16):
  zero16 = jnp.zeros((16,), jnp.float32)

  def zh(i, carry):
    ref[pl.ds(i * 16, 16)] = zero16
    return carry

  lax.fori_loop(0, n16, zh, 0)


def _prop_pipeline(table, srcv, dstv, rows0, rows1, acc, sem0, sem1,
                   scalar_work):
  def wait_gather(rows, sem):
    pltpu.make_async_copy(table.at[pl.ds(0, G)], rows, sem).wait()

  pltpu.async_copy(table.at[srcv.at[0]], rows0, sem0)

  def pair_body(gg, carry):
    g0 = 2 * gg
    pltpu.async_copy(table.at[srcv.at[g0 + 1]], rows1, sem1)
    scalar_work(g0)
    wait_gather(rows0, sem0)
    pltpu.sync_copy(rows0, acc.at[dstv.at[g0]], add=True)
    pltpu.async_copy(table.at[srcv.at[g0 + 2]], rows0, sem0)
    scalar_work(g0 + 1)
    wait_gather(rows1, sem1)
    pltpu.sync_copy(rows1, acc.at[dstv.at[g0 + 1]], add=True)
    return carry

  lax.fori_loop(0, (NG - 1) // 2, pair_body, 0)
  scalar_work(NG - 1)
  wait_gather(rows0, sem0)
  pltpu.sync_copy(rows0, acc.at[dstv.at[NG - 1]], add=True)


@functools.partial(
    pl.kernel,
    out_type=jax.ShapeDtypeStruct((NC, NPAD), jnp.float32),
    mesh=_MESH,
    compiler_params=_SC_PARAMS,
    scratch_types=[
        pltpu.VMEM((NC, NG, G), jnp.int32),
        pltpu.VMEM((NPAD,), jnp.float32),
        pltpu.VMEM((NS, RPT), jnp.float32),
        pltpu.VMEM_SHARED((NS, NPAD), jnp.float32),
    ],
)
def _pass1a(dst_h, dinv_h, dstH_v, hist, rbuf, shared):
  cid = lax.axis_index("c")
  sid = lax.axis_index("s")
  ones16 = jnp.full((16,), 1.0, jnp.float32)

  _zero_ref16(hist, NPAD // 16)
  pltpu.sync_copy(dst_h.at[pl.ds(NC * sid, NC)], dstH_v)

  def h_body(g, carry):
    for a in range(NC):
      for j in range(G // 16):
        idx = dstH_v[a, g, pl.ds(j * 16, 16)]
        plsc.addupdate_scatter(hist, [idx], ones16)
    return carry

  lax.fori_loop(0, NG, h_body, 0)

  pltpu.sync_copy(hist, shared.at[sid])
  plsc.subcore_barrier()
  pltpu.sync_copy(shared.at[:, pl.ds(sid * RPT, RPT)], rbuf)

  def r_body(cv, carry):
    a = rbuf[0, pl.ds(cv * 16, 16)]
    for r in range(1, NS):
      a = a + rbuf[r, pl.ds(cv * 16, 16)]
    x = a + 1.0
    hist[pl.ds(cv * 16, 16)] = _rsqrt16(x)
    return carry

  lax.fori_loop(0, RPT // 16, r_body, 0)
  pltpu.sync_copy(hist.at[pl.ds(0, RPT)],
                  dinv_h.at[cid, pl.ds(sid * RPT, RPT)])


@functools.partial(
    pl.kernel,
    out_type=[
        jax.ShapeDtypeStruct((NC, NPAD, 2, DH), jnp.float32),
        jax.ShapeDtypeStruct((NW, NPAD), jnp.float32),
        jax.ShapeDtypeStruct((NC, 2 * NPAD, DH), jnp.float32),
    ],
    mesh=_MESH,
    compiler_params=_SC_PARAMS,
    scratch_types=[
        pltpu.VMEM((NG, G), jnp.int32),
        pltpu.VMEM((NG, G), jnp.int32),
        pltpu.VMEM((G, DH), jnp.float32),
        pltpu.VMEM((G, DH), jnp.float32),
        pltpu.VMEM((ZR, DH), jnp.float32),
        pltpu.VMEM((PZ, DH), jnp.float32),
        pltpu.VMEM((PZ, DH), jnp.float32),
        pltpu.VMEM((NPAD,), jnp.float32),
        pltpu.VMEM((NPAD,), jnp.float32),
        pltpu.VMEM_SHARED((NPAD, DH), jnp.float32),
        pltpu.SemaphoreType.DMA,
        pltpu.SemaphoreType.DMA,
    ],
)
def _pass1b(srcs_h, dst_h, emb_h, dinv_h, parts_h, sparts_h, p0t_h,
            srcv, dstv, rows0, rows1, zbuf, embbuf0, embbuf1, dvecv, hist,
            acc, sem0, sem1):
  cid = lax.axis_index("c")
  sid = lax.axis_index("s")
  wid = cid * NS + sid
  zero16 = jnp.zeros((16,), jnp.float32)

  def zb(i, carry):
    for j in range(DH // 16):
      zbuf[i, pl.ds(j * 16, 16)] = zero16
    return carry

  lax.fori_loop(0, ZR, zb, 0)
  pltpu.sync_copy(dinv_h.at[cid], dvecv)

  base0 = 2 * sid * RPT
  nkc = 2 * RPT // PZ
  bufs = (embbuf0, embbuf1)
  sems = (sem0, sem1)
  pltpu.async_copy(emb_h.at[pl.ds(base0, PZ)], embbuf0, sem0)
  for k in range(nkc):
    b = k % 2
    buf, sem = bufs[b], sems[b]
    base = base0 + k * PZ
    if k + 1 < nkc:
      pltpu.async_copy(emb_h.at[pl.ds(base + PZ, PZ)], bufs[1 - b],
                       sems[1 - b])
    pltpu.make_async_copy(emb_h.at[pl.ds(0, PZ)], buf, sem).wait()

    def prow(n, carry, buf=buf, base=base):
      node16 = jnp.full((16,), base // 2 + n, jnp.int32)
      dv = plsc.load_gather(dvecv, [node16])
      for rr in range(2):
        for c in range(DH // 16):
          sl = pl.ds(c * 16, 16)
          buf[2 * n + rr, sl] = buf[2 * n + rr, sl] * dv
      return carry

    lax.fori_loop(0, PZ // 2, prow, 0)
    pltpu.sync_copy(buf, p0t_h.at[cid, pl.ds(base, PZ)])

  _zero_ref16(hist, NPAD // 16)
  for k in range(RPT // ZR):
    pltpu.sync_copy(zbuf, acc.at[pl.ds(sid * RPT + k * ZR, ZR)])
  pltpu.sync_copy(dst_h.at[wid], dstv)
  plsc.subcore_barrier()

  for h in range(2):
    pltpu.sync_copy(srcs_h.at[h, wid], srcv)
    if h == 1:
      plsc.subcore_barrier()

    if h == 0:
      def scalar_work(g):
        for j in range(G // 16):
          sidx = lax.shift_right_logical(srcv[g, pl.ds(j * 16, 16)], 1)
          didx = dstv[g, pl.ds(j * 16, 16)]
          vals = plsc.load_gather(dvecv, [sidx])
          plsc.addupdate_scatter(hist, [didx], vals)
    else:
      def scalar_work(g):
        del g

    _prop_pipeline(p0t_h.at[cid], srcv, dstv, rows0, rows1, acc, sem0, sem1,
                   scalar_work)
    plsc.subcore_barrier()
    pltpu.sync_copy(acc.at[pl.ds(sid * RPT, RPT)],
                    parts_h.at[cid, pl.ds(sid * RPT, RPT), h])
    if h == 0:
      for k in range(RPT // ZR):
        pltpu.sync_copy(zbuf, acc.at[pl.ds(sid * RPT + k * ZR, ZR)])
  pltpu.sync_copy(hist, sparts_h.at[wid])


@functools.partial(
    pl.kernel,
    out_type=[
        jax.ShapeDtypeStruct((NC, NPAD, 2, DH), jnp.float32),
        jax.ShapeDtypeStruct((NC, 2 * NPAD, DH), jnp.float32),
    ],
    mesh=_MESH,
    compiler_params=_SC_PARAMS,
    scratch_types=[
        pltpu.VMEM((NG, G), jnp.int32),
        pltpu.VMEM((NG, G), jnp.int32),
        pltpu.VMEM((G, DH), jnp.float32),
        pltpu.VMEM((G, DH), jnp.float32),
        pltpu.VMEM((ZR, DH), jnp.float32),
        pltpu.VMEM((RPT,), jnp.float32),
        pltpu.VMEM((2, PB, 2, DH), jnp.float32),
        pltpu.VMEM((2, PB, 2, DH), jnp.float32),
        pltpu.VMEM((2, 2 * PB, DH), jnp.float32),
        pltpu.VMEM_SHARED((NPAD, DH), jnp.float32),
        pltpu.SemaphoreType.DMA,
        pltpu.SemaphoreType.DMA,
        pltpu.SemaphoreType.DMA,
        pltpu.SemaphoreType.DMA,
    ],
)
def _pass2(srcs_h, dst_h, dinv_h, parts0_h, p0t_h, parts_h, p1t_h,
           srcv, dstv, rows0, rows1, zbuf, dloc, bufA, bufB, pbuf, acc,
           sem0, sem1, sem2, sem3):
  cid = lax.axis_index("c")
  sid = lax.axis_index("s")
  wid = cid * NS + sid
  zero16 = jnp.zeros((16,), jnp.float32)

  def zb(i, carry):
    for j in range(DH // 16):
      zbuf[i, pl.ds(j * 16, 16)] = zero16
    return carry

  lax.fori_loop(0, ZR, zb, 0)

  pltpu.sync_copy(dinv_h.at[cid, pl.ds(sid * RPT, RPT)], dloc)

  nkb = RPT // PB
  bsems = (sem0, sem1)

  def issue_reads(k, b):
    nb = sid * RPT + k * PB
    pltpu.async_copy(parts0_h.at[0, pl.ds(nb, PB)], bufA.at[b], bsems[b])
    pltpu.async_copy(parts0_h.at[1, pl.ds(nb, PB)], bufB.at[b], bsems[b])
    pltpu.async_copy(p0t_h.at[cid, pl.ds(2 * nb, 2 * PB)], pbuf.at[b],
                     bsems[b])

  issue_reads(0, 0)
  for k in range(nkb):
    b = k % 2
    if k + 1 < nkb:
      issue_reads(k + 1, 1 - b)
    pltpu.make_async_copy(parts0_h.at[0, pl.ds(0, PB)], bufA.at[b],
                          bsems[b]).wait()
    pltpu.make_async_copy(parts0_h.at[0, pl.ds(0, PB)], bufB.at[b],
                          bsems[b]).wait()
    pltpu.make_async_copy(p0t_h.at[cid, pl.ds(0, 2 * PB)], pbuf.at[b],
                          bsems[b]).wait()
    nb = sid * RPT + k * PB
    lb = k * PB

    def prow(r, carry, b=b, lb=lb):
      l16 = jnp.full((16,), lb + r, jnp.int32)
      dv = plsc.load_gather(dloc, [l16])
      dv2 = dv * dv
      for hh in range(2):
        for c in range(DH // 16):
          sl = pl.ds(c * 16, 16)
          pbuf[b, 2 * r + hh, sl] = (
              bufA[b, r, hh, sl] + bufB[b, r, hh, sl]
              + pbuf[b, 2 * r + hh, sl]) * dv2
      return carry

    lax.fori_loop(0, PB, prow, 0)
    pltpu.sync_copy(pbuf.at[b], p1t_h.at[cid, pl.ds(2 * nb, 2 * PB)])

  for k in range(RPT // ZR):
    pltpu.sync_copy(zbuf, acc.at[pl.ds(sid * RPT + k * ZR, ZR)])
  pltpu.sync_copy(dst_h.at[wid], dstv)
  plsc.subcore_barrier()

  def no_scalar(g):
    del g

  for h in range(2):
    pltpu.sync_copy(srcs_h.at[h, wid], srcv)
    if h == 1:
      plsc.subcore_barrier()
    _prop_pipeline(p1t_h.at[cid], srcv, dstv, rows0, rows1, acc, sem2, sem3,
                   no_scalar)
    plsc.subcore_barrier()
    pltpu.sync_copy(acc.at[pl.ds(sid * RPT, RPT)],
                    parts_h.at[cid, pl.ds(sid * RPT, RPT), h])
    if h == 0:
      for k in range(RPT // ZR):
        pltpu.sync_copy(zbuf, acc.at[pl.ds(sid * RPT + k * ZR, ZR)])


_RB = 1024


def _final_body(parts_ref, p1_ref, dinv_ref, sparts_ref,
                w1_ref, b1_ref, w2_ref, b2_ref, out_ref):
  d = dinv_ref[0]
  p1r = p1_ref[0].reshape(_RB, 2, DH)
  q2 = (parts_ref[0] + parts_ref[1] + p1r) * d[:, None, None]
  w12 = jnp.dot(w1_ref[...], w2_ref[...], preferred_element_type=jnp.float32)
  b12 = jnp.dot(b1_ref[...][None, :], w2_ref[...],
                preferred_element_type=jnp.float32)[0]
  s = d * (jnp.sum(sparts_ref[...], axis=0) + d)
  out_ref[...] = (
      jnp.dot(q2[:, 0, :], w12[:DH, :], preferred_element_type=jnp.float32)
      + jnp.dot(q2[:, 1, :], w12[DH:, :], preferred_element_type=jnp.float32)
      + s[:, None] * b12[None, :] + b2_ref[...][None, :])


def _final_call(parts1, p1t, dinv, sparts, W1, b1, W2, b2):
  return pl.pallas_call(
      _final_body,
      grid=(NPAD // _RB,),
      in_specs=[
          pl.BlockSpec((NC, _RB, 2, DH), lambda i: (0, i, 0, 0)),
          pl.BlockSpec((1, 2 * _RB, DH), lambda i: (0, i, 0)),
          pl.BlockSpec((NC, _RB), lambda i: (0, i)),
          pl.BlockSpec((NW, _RB), lambda i: (0, i)),
          pl.BlockSpec((D, 2 * D), lambda i: (0, 0)),
          pl.BlockSpec((2 * D,), lambda i: (0,)),
          pl.BlockSpec((2 * D, D), lambda i: (0, 0)),
          pl.BlockSpec((D,), lambda i: (0,)),
      ],
      out_specs=pl.BlockSpec((_RB, D), lambda i: (i, 0)),
      out_shape=jax.ShapeDtypeStruct((NPAD, D), jnp.float32),
  )(parts1, p1t, dinv, sparts, W1, b1, W2, b2)


def kernel(edge_index, edge_weight, emb, W1, b1, W2, b2):
  src = edge_index[0].astype(jnp.int32)
  dst = edge_index[1].astype(jnp.int32)
  src2 = src * 2
  srcs = jnp.stack([src2, src2 + 1]).reshape(2, NW, NG, G)
  dst3 = dst.reshape(NW, NG, G)
  emb2 = jnp.pad(emb, ((0, NPAD - N_NODES), (0, 0))).reshape(2 * NPAD, DH)

  dinv = _pass1a(dst3)
  parts0, sparts, p0t = _pass1b(srcs, dst3, emb2, dinv)
  parts1, p1t = _pass2(srcs, dst3, dinv, parts0, p0t)
  out = _final_call(parts1, p1t, dinv, sparts, W1, b1, W2, b2)
  return out[:N_NODES]


# speedup vs baseline: 1.1101x; 1.0103x over previous
import functools

import jax
import jax.numpy as jnp
from jax import lax
from jax.experimental import pallas as pl
from jax.experimental.pallas import tpu as pltpu
from jax.experimental.pallas import tpu_sc as plsc

N_NODES = 10000
D = 128
DH = D // 2
E = 320000

NC = 2
NS = 16
NW = NC * NS
NPAD = 10240
G = 80
EPT = E // NW
NG = EPT // G
NGH = E // NS // G
RPT = NPAD // NS
ZR = 80
PZ = 160
PB = 40

_MESH = plsc.VectorSubcoreMesh(core_axis_name="c", subcore_axis_name="s")
_SC_PARAMS = pltpu.CompilerParams(needs_layout_passes=False,
                                  use_tc_tiling_on_sc=False)


def _rsqrt16(x):
  i = plsc.bitcast(x, jnp.int32)
  i = jnp.int32(0x5F3759DF) - lax.shift_right_logical(i, 1)
  y = plsc.bitcast(i, jnp.float32)
  for _ in range(3):
    y = y * (1.5 - 0.5 * x * y * y)
  return y


def _zero_ref16(ref, n16):
  zero16 = jnp.zeros((16,), jnp.float32)

  def zh(i, carry):
    ref[pl.ds(i * 16, 16)] = zero16
    return carry

  lax.fori_loop(0, n16, zh, 0)


def _prop_pipeline(table, srcv, dstv, rows0, rows1, acc, sem0, sem1,
                   scalar_work):
  def wait_gather(rows, sem):
    pltpu.make_async_copy(table.at[pl.ds(0, G)], rows, sem).wait()

  pltpu.async_copy(table.at[srcv.at[0]], rows0, sem0)

  def pair_body(gg, carry):
    g0 = 2 * gg
    pltpu.async_copy(table.at[srcv.at[g0 + 1]], rows1, sem1)
    scalar_work(g0)
    wait_gather(rows0, sem0)
    pltpu.sync_copy(rows0, acc.at[dstv.at[g0]], add=True)
    pltpu.async_copy(table.at[srcv.at[g0 + 2]], rows0, sem0)
    scalar_work(g0 + 1)
    wait_gather(rows1, sem1)
    pltpu.sync_copy(rows1, acc.at[dstv.at[g0 + 1]], add=True)
    return carry

  lax.fori_loop(0, (NG - 1) // 2, pair_body, 0)
  scalar_work(NG - 1)
  wait_gather(rows0, sem0)
  pltpu.sync_copy(rows0, acc.at[dstv.at[NG - 1]], add=True)


@functools.partial(
    pl.kernel,
    out_type=jax.ShapeDtypeStruct((NC, NPAD), jnp.float32),
    mesh=_MESH,
    compiler_params=_SC_PARAMS,
    scratch_types=[
        pltpu.VMEM((NC, NG, G), jnp.int32),
        pltpu.VMEM((NPAD,), jnp.float32),
        pltpu.VMEM((NS, RPT), jnp.float32),
        pltpu.VMEM_SHARED((NS, NPAD), jnp.float32),
    ],
)
def _pass1a(dst_h, dinv_h, dstH_v, hist, rbuf, shared):
  cid = lax.axis_index("c")
  sid = lax.axis_index("s")
  ones16 = jnp.full((16,), 1.0, jnp.float32)

  _zero_ref16(hist, NPAD // 16)
  pltpu.sync_copy(dst_h.at[pl.ds(NC * sid, NC)], dstH_v)

  def h_body(g, carry):
    for a in range(NC):
      for j in range(G // 16):
        idx = dstH_v[a, g, pl.ds(j * 16, 16)]
        plsc.addupdate_scatter(hist, [idx], ones16)
    return carry

  lax.fori_loop(0, NG, h_body, 0)

  pltpu.sync_copy(hist, shared.at[sid])
  plsc.subcore_barrier()
  pltpu.sync_copy(shared.at[:, pl.ds(sid * RPT, RPT)], rbuf)

  def r_body(cv, carry):
    a = rbuf[0, pl.ds(cv * 16, 16)]
    for r in range(1, NS):
      a = a + rbuf[r, pl.ds(cv * 16, 16)]
    x = a + 1.0
    hist[pl.ds(cv * 16, 16)] = _rsqrt16(x)
    return carry

  lax.fori_loop(0, RPT // 16, r_body, 0)
  pltpu.sync_copy(hist.at[pl.ds(0, RPT)],
                  dinv_h.at[cid, pl.ds(sid * RPT, RPT)])


@functools.partial(
    pl.kernel,
    out_type=[
        jax.ShapeDtypeStruct((NC, NPAD, 2, DH), jnp.float32),
        jax.ShapeDtypeStruct((NW, NPAD), jnp.float32),
        jax.ShapeDtypeStruct((NC, 2 * NPAD, DH), jnp.float32),
    ],
    mesh=_MESH,
    compiler_params=_SC_PARAMS,
    scratch_types=[
        pltpu.VMEM((NG, G), jnp.int32),
        pltpu.VMEM((NG, G), jnp.int32),
        pltpu.VMEM((G, DH), jnp.float32),
        pltpu.VMEM((G, DH), jnp.float32),
        pltpu.VMEM((ZR, DH), jnp.float32),
        pltpu.VMEM((PZ, DH), jnp.float32),
        pltpu.VMEM((PZ, DH), jnp.float32),
        pltpu.VMEM((NPAD,), jnp.float32),
        pltpu.VMEM((NPAD,), jnp.float32),
        pltpu.VMEM_SHARED((NPAD, DH), jnp.float32),
        pltpu.SemaphoreType.DMA,
        pltpu.SemaphoreType.DMA,
    ],
)
def _pass1b(srcs_h, dst_h, emb_h, dinv_h, parts_h, sparts_h, p0t_h,
            srcv, dstv, rows0, rows1, zbuf, embbuf0, embbuf1, dvecv, hist,
            acc, sem0, sem1):
  cid = lax.axis_index("c")
  sid = lax.axis_index("s")
  wid = cid * NS + sid
  zero16 = jnp.zeros((16,), jnp.float32)

  def zb(i, carry):
    for j in range(DH // 16):
      zbuf[i, pl.ds(j * 16, 16)] = zero16
    return carry

  lax.fori_loop(0, ZR, zb, 0)
  pltpu.sync_copy(dinv_h.at[cid], dvecv)

  base0 = 2 * sid * RPT
  nkc = 2 * RPT // PZ
  bufs = (embbuf0, embbuf1)
  sems = (sem0, sem1)
  pltpu.async_copy(emb_h.at[pl.ds(base0, PZ)], embbuf0, sem0)
  for k in range(nkc):
    b = k % 2
    buf, sem = bufs[b], sems[b]
    base = base0 + k * PZ
    if k + 1 < nkc:
      pltpu.async_copy(emb_h.at[pl.ds(base + PZ, PZ)], bufs[1 - b],
                       sems[1 - b])
    pltpu.make_async_copy(emb_h.at[pl.ds(0, PZ)], buf, sem).wait()

    def prow(n, carry, buf=buf, base=base):
      node16 = jnp.full((16,), base // 2 + n, jnp.int32)
      dv = plsc.load_gather(dvecv, [node16])
      for rr in range(2):
        for c in range(DH // 16):
          sl = pl.ds(c * 16, 16)
          buf[2 * n + rr, sl] = buf[2 * n + rr, sl] * dv
      return carry

    lax.fori_loop(0, PZ // 2, prow, 0)
    pltpu.sync_copy(buf, p0t_h.at[cid, pl.ds(base, PZ)])

  _zero_ref16(hist, NPAD // 16)
  for k in range(RPT // ZR):
    pltpu.sync_copy(zbuf, acc.at[pl.ds(sid * RPT + k * ZR, ZR)])
  pltpu.sync_copy(dst_h.at[wid], dstv)
  plsc.subcore_barrier()

  for h in range(2):
    pltpu.sync_copy(srcs_h.at[h, wid], srcv)
    if h == 1:
      plsc.subcore_barrier()

    if h == 0:
      def scalar_work(g):
        for j in range(G // 16):
          sidx = lax.shift_right_logical(srcv[g, pl.ds(j * 16, 16)], 1)
          didx = dstv[g, pl.ds(j * 16, 16)]
          vals = plsc.load_gather(dvecv, [sidx])
          plsc.addupdate_scatter(hist, [didx], vals)
    else:
      def scalar_work(g):
        del g

    _prop_pipeline(p0t_h.at[cid], srcv, dstv, rows0, rows1, acc, sem0, sem1,
                   scalar_work)
    plsc.subcore_barrier()
    pltpu.sync_copy(acc.at[pl.ds(sid * RPT, RPT)],
                    parts_h.at[cid, pl.ds(sid * RPT, RPT), h])
    if h == 0:
      for k in range(RPT // ZR):
        pltpu.sync_copy(zbuf, acc.at[pl.ds(sid * RPT + k * ZR, ZR)])
  pltpu.sync_copy(hist, sparts_h.at[wid])


@functools.partial(
    pl.kernel,
    out_type=[
        jax.ShapeDtypeStruct((NC, NPAD, 2, DH), jnp.float32),
        jax.ShapeDtypeStruct((NC, 2 * NPAD, DH), jnp.float32),
    ],
    mesh=_MESH,
    compiler_params=_SC_PARAMS,
    scratch_types=[
        pltpu.VMEM((NG, G), jnp.int32),
        pltpu.VMEM((NG, G), jnp.int32),
        pltpu.VMEM((G, DH), jnp.float32),
        pltpu.VMEM((G, DH), jnp.float32),
        pltpu.VMEM((ZR, DH), jnp.float32),
        pltpu.VMEM((RPT,), jnp.float32),
        pltpu.VMEM((2, PB, 2, DH), jnp.float32),
        pltpu.VMEM((2, PB, 2, DH), jnp.float32),
        pltpu.VMEM((2, 2 * PB, DH), jnp.float32),
        pltpu.VMEM_SHARED((NPAD, DH), jnp.float32),
        pltpu.SemaphoreType.DMA,
        pltpu.SemaphoreType.DMA,
        pltpu.SemaphoreType.DMA,
        pltpu.SemaphoreType.DMA,
    ],
)
def _pass2(srcs_h, dst_h, dinv_h, parts0_h, p0t_h, parts_h, p1t_h,
           srcv, dstv, rows0, rows1, zbuf, dloc, bufA, bufB, pbuf, acc,
           sem0, sem1, sem2, sem3):
  cid = lax.axis_index("c")
  sid = lax.axis_index("s")
  wid = cid * NS + sid
  zero16 = jnp.zeros((16,), jnp.float32)

  def zb(i, carry):
    for j in range(DH // 16):
      zbuf[i, pl.ds(j * 16, 16)] = zero16
    return carry

  lax.fori_loop(0, ZR, zb, 0)

  pltpu.sync_copy(dinv_h.at[cid, pl.ds(sid * RPT, RPT)], dloc)

  nkb = RPT // PB
  bsems = (sem0, sem1)

  def issue_reads(k, b):
    nb = sid * RPT + k * PB
    pltpu.async_copy(parts0_h.at[0, pl.ds(nb, PB)], bufA.at[b], bsems[b])
    pltpu.async_copy(parts0_h.at[1, pl.ds(nb, PB)], bufB.at[b], bsems[b])
    pltpu.async_copy(p0t_h.at[cid, pl.ds(2 * nb, 2 * PB)], pbuf.at[b],
                     bsems[b])

  issue_reads(0, 0)
  for k in range(nkb):
    b = k % 2
    if k + 1 < nkb:
      issue_reads(k + 1, 1 - b)
    pltpu.make_async_copy(parts0_h.at[0, pl.ds(0, PB)], bufA.at[b],
                          bsems[b]).wait()
    pltpu.make_async_copy(parts0_h.at[0, pl.ds(0, PB)], bufB.at[b],
                          bsems[b]).wait()
    pltpu.make_async_copy(p0t_h.at[cid, pl.ds(0, 2 * PB)], pbuf.at[b],
                          bsems[b]).wait()
    nb = sid * RPT + k * PB
    lb = k * PB

    def prow(r, carry, b=b, lb=lb):
      l16 = jnp.full((16,), lb + r, jnp.int32)
      dv = plsc.load_gather(dloc, [l16])
      dv2 = dv * dv
      for hh in range(2):
        for c in range(DH // 16):
          sl = pl.ds(c * 16, 16)
          pbuf[b, 2 * r + hh, sl] = (
              bufA[b, r, hh, sl] + bufB[b, r, hh, sl]
              + pbuf[b, 2 * r + hh, sl]) * dv2
      return carry

    lax.fori_loop(0, PB, prow, 0)
    pltpu.sync_copy(pbuf.at[b], p1t_h.at[cid, pl.ds(2 * nb, 2 * PB)])

  for k in range(RPT // ZR):
    pltpu.sync_copy(zbuf, acc.at[pl.ds(sid * RPT + k * ZR, ZR)])
  pltpu.sync_copy(dst_h.at[wid], dstv)
  plsc.subcore_barrier()

  def no_scalar(g):
    del g

  for h in range(2):
    pltpu.sync_copy(srcs_h.at[h, wid], srcv)
    if h == 1:
      plsc.subcore_barrier()
    _prop_pipeline(p1t_h.at[cid], srcv, dstv, rows0, rows1, acc, sem2, sem3,
                   no_scalar)
    plsc.subcore_barrier()
    pltpu.sync_copy(acc.at[pl.ds(sid * RPT, RPT)],
                    parts_h.at[cid, pl.ds(sid * RPT, RPT), h])
    if h == 0:
      for k in range(RPT // ZR):
        pltpu.sync_copy(zbuf, acc.at[pl.ds(sid * RPT + k * ZR, ZR)])


_RB = 1024


def _final_body(parts_ref, p1_ref, dinv_ref, sparts_ref,
                w1_ref, b1_ref, w2_ref, b2_ref, out_ref):
  d = dinv_ref[0]
  p1r = p1_ref[0].reshape(_RB, 2, DH)
  q2 = (parts_ref[0] + parts_ref[1] + p1r) * d[:, None, None]
  w12 = jnp.dot(w1_ref[...], w2_ref[...], preferred_element_type=jnp.float32)
  b12 = jnp.dot(b1_ref[...][None, :], w2_ref[...],
                preferred_element_type=jnp.float32)[0]
  s = d * (jnp.sum(sparts_ref[...], axis=0) + d)
  out_ref[...] = (
      jnp.dot(q2[:, 0, :], w12[:DH, :], preferred_element_type=jnp.float32)
      + jnp.dot(q2[:, 1, :], w12[DH:, :], preferred_element_type=jnp.float32)
      + s[:, None] * b12[None, :] + b2_ref[...][None, :])


def _final_call(parts1, p1t, dinv, sparts, W1, b1, W2, b2):
  return pl.pallas_call(
      _final_body,
      grid=(NPAD // _RB,),
      in_specs=[
          pl.BlockSpec((NC, _RB, 2, DH), lambda i: (0, i, 0, 0)),
          pl.BlockSpec((1, 2 * _RB, DH), lambda i: (0, i, 0)),
          pl.BlockSpec((NC, _RB), lambda i: (0, i)),
          pl.BlockSpec((NW, _RB), lambda i: (0, i)),
          pl.BlockSpec((D, 2 * D), lambda i: (0, 0)),
          pl.BlockSpec((2 * D,), lambda i: (0,)),
          pl.BlockSpec((2 * D, D), lambda i: (0, 0)),
          pl.BlockSpec((D,), lambda i: (0,)),
      ],
      out_specs=pl.BlockSpec((_RB, D), lambda i: (i, 0)),
      out_shape=jax.ShapeDtypeStruct((NPAD, D), jnp.float32),
  )(parts1, p1t, dinv, sparts, W1, b1, W2, b2)


def kernel(edge_index, edge_weight, emb, W1, b1, W2, b2):
  src = edge_index[0].astype(jnp.int32)
  dst = edge_index[1].astype(jnp.int32)
  src2 = src * 2
  srcs = jnp.stack([src2, src2 + 1]).reshape(2, NW, NG, G)
  dst3 = dst.reshape(NW, NG, G)
  emb2 = jnp.pad(emb, ((0, NPAD - N_NODES), (0, 0))).reshape(2 * NPAD, DH)

  dinv = _pass1a(dst3)
  parts0, sparts, p0t = _pass1b(srcs, dst3, emb2, dinv)
  parts1, p1t = _pass2(srcs, dst3, dinv, parts0, p0t)
  out = _final_call(parts1, p1t, dinv, sparts, W1, b1, W2, b2)
  return out[:N_NODES]


# speedup vs baseline: 1.1148x; 1.0042x over previous
import functools

import jax
import jax.numpy as jnp
from jax import lax
from jax.experimental import pallas as pl
from jax.experimental.pallas import tpu as pltpu
from jax.experimental.pallas import tpu_sc as plsc

N_NODES = 10000
D = 128
DH = D // 2
E = 320000

NC = 2
NS = 16
NW = NC * NS
NPAD = 10240
G = 80
EPT = E // NW
NG = EPT // G
NGH = E // NS // G
RPT = NPAD // NS
ZR = 80
PZ = 160
PB = 64

_MESH = plsc.VectorSubcoreMesh(core_axis_name="c", subcore_axis_name="s")
_SC_PARAMS = pltpu.CompilerParams(needs_layout_passes=False,
                                  use_tc_tiling_on_sc=False)


def _rsqrt16(x):
  i = plsc.bitcast(x, jnp.int32)
  i = jnp.int32(0x5F3759DF) - lax.shift_right_logical(i, 1)
  y = plsc.bitcast(i, jnp.float32)
  for _ in range(3):
    y = y * (1.5 - 0.5 * x * y * y)
  return y


def _zero_ref16(ref, n16):
  zero16 = jnp.zeros((16,), jnp.float32)

  def zh(i, carry):
    ref[pl.ds(i * 16, 16)] = zero16
    return carry

  lax.fori_loop(0, n16, zh, 0)


def _prop_pipeline(table, srcv, dstv, rows0, rows1, acc, sem0, sem1,
                   scalar_work):
  def wait_gather(rows, sem):
    pltpu.make_async_copy(table.at[pl.ds(0, G)], rows, sem).wait()

  pltpu.async_copy(table.at[srcv.at[0]], rows0, sem0)

  def pair_body(gg, carry):
    g0 = 2 * gg
    pltpu.async_copy(table.at[srcv.at[g0 + 1]], rows1, sem1)
    scalar_work(g0)
    wait_gather(rows0, sem0)
    pltpu.sync_copy(rows0, acc.at[dstv.at[g0]], add=True)
    pltpu.async_copy(table.at[srcv.at[g0 + 2]], rows0, sem0)
    scalar_work(g0 + 1)
    wait_gather(rows1, sem1)
    pltpu.sync_copy(rows1, acc.at[dstv.at[g0 + 1]], add=True)
    return carry

  lax.fori_loop(0, (NG - 1) // 2, pair_body, 0)
  scalar_work(NG - 1)
  wait_gather(rows0, sem0)
  pltpu.sync_copy(rows0, acc.at[dstv.at[NG - 1]], add=True)


@functools.partial(
    pl.kernel,
    out_type=jax.ShapeDtypeStruct((NC, NPAD), jnp.float32),
    mesh=_MESH,
    compiler_params=_SC_PARAMS,
    scratch_types=[
        pltpu.VMEM((NC, NG, G), jnp.int32),
        pltpu.VMEM((NPAD,), jnp.float32),
        pltpu.VMEM((NS, RPT), jnp.float32),
        pltpu.VMEM_SHARED((NS, NPAD), jnp.float32),
    ],
)
def _pass1a(dst_h, dinv_h, dstH_v, hist, rbuf, shared):
  cid = lax.axis_index("c")
  sid = lax.axis_index("s")
  ones16 = jnp.full((16,), 1.0, jnp.float32)

  _zero_ref16(hist, NPAD // 16)
  pltpu.sync_copy(dst_h.at[pl.ds(NC * sid, NC)], dstH_v)

  def h_body(g, carry):
    for a in range(NC):
      for j in range(G // 16):
        idx = dstH_v[a, g, pl.ds(j * 16, 16)]
        plsc.addupdate_scatter(hist, [idx], ones16)
    return carry

  lax.fori_loop(0, NG, h_body, 0)

  pltpu.sync_copy(hist, shared.at[sid])
  plsc.subcore_barrier()
  pltpu.sync_copy(shared.at[:, pl.ds(sid * RPT, RPT)], rbuf)

  def r_body(cv, carry):
    a = rbuf[0, pl.ds(cv * 16, 16)]
    for r in range(1, NS):
      a = a + rbuf[r, pl.ds(cv * 16, 16)]
    x = a + 1.0
    hist[pl.ds(cv * 16, 16)] = _rsqrt16(x)
    return carry

  lax.fori_loop(0, RPT // 16, r_body, 0)
  pltpu.sync_copy(hist.at[pl.ds(0, RPT)],
                  dinv_h.at[cid, pl.ds(sid * RPT, RPT)])


@functools.partial(
    pl.kernel,
    out_type=[
        jax.ShapeDtypeStruct((NC, NPAD, 2, DH), jnp.float32),
        jax.ShapeDtypeStruct((NW, NPAD), jnp.float32),
        jax.ShapeDtypeStruct((NC, 2 * NPAD, DH), jnp.float32),
    ],
    mesh=_MESH,
    compiler_params=_SC_PARAMS,
    scratch_types=[
        pltpu.VMEM((NG, G), jnp.int32),
        pltpu.VMEM((NG, G), jnp.int32),
        pltpu.VMEM((G, DH), jnp.float32),
        pltpu.VMEM((G, DH), jnp.float32),
        pltpu.VMEM((ZR, DH), jnp.float32),
        pltpu.VMEM((PZ, DH), jnp.float32),
        pltpu.VMEM((PZ, DH), jnp.float32),
        pltpu.VMEM((NPAD,), jnp.float32),
        pltpu.VMEM((NPAD,), jnp.float32),
        pltpu.VMEM_SHARED((NPAD, DH), jnp.float32),
        pltpu.SemaphoreType.DMA,
        pltpu.SemaphoreType.DMA,
    ],
)
def _pass1b(srcs_h, dst_h, emb_h, dinv_h, parts_h, sparts_h, p0t_h,
            srcv, dstv, rows0, rows1, zbuf, embbuf0, embbuf1, dvecv, hist,
            acc, sem0, sem1):
  cid = lax.axis_index("c")
  sid = lax.axis_index("s")
  wid = cid * NS + sid
  zero16 = jnp.zeros((16,), jnp.float32)

  def zb(i, carry):
    for j in range(DH // 16):
      zbuf[i, pl.ds(j * 16, 16)] = zero16
    return carry

  lax.fori_loop(0, ZR, zb, 0)
  pltpu.sync_copy(dinv_h.at[cid], dvecv)

  base0 = 2 * sid * RPT
  nkc = 2 * RPT // PZ
  bufs = (embbuf0, embbuf1)
  sems = (sem0, sem1)
  pltpu.async_copy(emb_h.at[pl.ds(base0, PZ)], embbuf0, sem0)
  for k in range(nkc):
    b = k % 2
    buf, sem = bufs[b], sems[b]
    base = base0 + k * PZ
    if k + 1 < nkc:
      pltpu.async_copy(emb_h.at[pl.ds(base + PZ, PZ)], bufs[1 - b],
                       sems[1 - b])
    pltpu.make_async_copy(emb_h.at[pl.ds(0, PZ)], buf, sem).wait()

    def prow(n, carry, buf=buf, base=base):
      node16 = jnp.full((16,), base // 2 + n, jnp.int32)
      dv = plsc.load_gather(dvecv, [node16])
      for rr in range(2):
        for c in range(DH // 16):
          sl = pl.ds(c * 16, 16)
          buf[2 * n + rr, sl] = buf[2 * n + rr, sl] * dv
      return carry

    lax.fori_loop(0, PZ // 2, prow, 0)
    pltpu.sync_copy(buf, p0t_h.at[cid, pl.ds(base, PZ)])

  _zero_ref16(hist, NPAD // 16)
  for k in range(RPT // ZR):
    pltpu.sync_copy(zbuf, acc.at[pl.ds(sid * RPT + k * ZR, ZR)])
  pltpu.sync_copy(dst_h.at[wid], dstv)
  plsc.subcore_barrier()

  for h in range(2):
    pltpu.sync_copy(srcs_h.at[h, wid], srcv)
    if h == 1:
      plsc.subcore_barrier()

    if h == 0:
      def scalar_work(g):
        for j in range(G // 16):
          sidx = lax.shift_right_logical(srcv[g, pl.ds(j * 16, 16)], 1)
          didx = dstv[g, pl.ds(j * 16, 16)]
          vals = plsc.load_gather(dvecv, [sidx])
          plsc.addupdate_scatter(hist, [didx], vals)
    else:
      def scalar_work(g):
        del g

    _prop_pipeline(p0t_h.at[cid], srcv, dstv, rows0, rows1, acc, sem0, sem1,
                   scalar_work)
    plsc.subcore_barrier()
    pltpu.sync_copy(acc.at[pl.ds(sid * RPT, RPT)],
                    parts_h.at[cid, pl.ds(sid * RPT, RPT), h])
    if h == 0:
      for k in range(RPT // ZR):
        pltpu.sync_copy(zbuf, acc.at[pl.ds(sid * RPT + k * ZR, ZR)])
  pltpu.sync_copy(hist, sparts_h.at[wid])


@functools.partial(
    pl.kernel,
    out_type=[
        jax.ShapeDtypeStruct((NC, NPAD, 2, DH), jnp.float32),
        jax.ShapeDtypeStruct((NC, 2 * NPAD, DH), jnp.float32),
    ],
    mesh=_MESH,
    compiler_params=_SC_PARAMS,
    scratch_types=[
        pltpu.VMEM((NG, G), jnp.int32),
        pltpu.VMEM((NG, G), jnp.int32),
        pltpu.VMEM((G, DH), jnp.float32),
        pltpu.VMEM((G, DH), jnp.float32),
        pltpu.VMEM((ZR, DH), jnp.float32),
        pltpu.VMEM((RPT,), jnp.float32),
        pltpu.VMEM((2, PB, 2, DH), jnp.float32),
        pltpu.VMEM((2, PB, 2, DH), jnp.float32),
        pltpu.VMEM((2, 2 * PB, DH), jnp.float32),
        pltpu.VMEM_SHARED((NPAD, DH), jnp.float32),
        pltpu.SemaphoreType.DMA,
        pltpu.SemaphoreType.DMA,
        pltpu.SemaphoreType.DMA,
        pltpu.SemaphoreType.DMA,
    ],
)
def _pass2(srcs_h, dst_h, dinv_h, parts0_h, p0t_h, parts_h, p1t_h,
           srcv, dstv, rows0, rows1, zbuf, dloc, bufA, bufB, pbuf, acc,
           sem0, sem1, sem2, sem3):
  cid = lax.axis_index("c")
  sid = lax.axis_index("s")
  wid = cid * NS + sid
  zero16 = jnp.zeros((16,), jnp.float32)

  def zb(i, carry):
    for j in range(DH // 16):
      zbuf[i, pl.ds(j * 16, 16)] = zero16
    return carry

  lax.fori_loop(0, ZR, zb, 0)

  pltpu.sync_copy(dinv_h.at[cid, pl.ds(sid * RPT, RPT)], dloc)

  nkb = RPT // PB
  bsems = (sem0, sem1)

  def issue_reads(k, b):
    nb = sid * RPT + k * PB
    pltpu.async_copy(parts0_h.at[0, pl.ds(nb, PB)], bufA.at[b], bsems[b])
    pltpu.async_copy(parts0_h.at[1, pl.ds(nb, PB)], bufB.at[b], bsems[b])
    pltpu.async_copy(p0t_h.at[cid, pl.ds(2 * nb, 2 * PB)], pbuf.at[b],
                     bsems[b])

  issue_reads(0, 0)
  for k in range(nkb):
    b = k % 2
    if k + 1 < nkb:
      issue_reads(k + 1, 1 - b)
    pltpu.make_async_copy(parts0_h.at[0, pl.ds(0, PB)], bufA.at[b],
                          bsems[b]).wait()
    pltpu.make_async_copy(parts0_h.at[0, pl.ds(0, PB)], bufB.at[b],
                          bsems[b]).wait()
    pltpu.make_async_copy(p0t_h.at[cid, pl.ds(0, 2 * PB)], pbuf.at[b],
                          bsems[b]).wait()
    nb = sid * RPT + k * PB
    lb = k * PB

    def prow(r, carry, b=b, lb=lb):
      l16 = jnp.full((16,), lb + r, jnp.int32)
      dv = plsc.load_gather(dloc, [l16])
      dv2 = dv * dv
      for hh in range(2):
        for c in range(DH // 16):
          sl = pl.ds(c * 16, 16)
          pbuf[b, 2 * r + hh, sl] = (
              bufA[b, r, hh, sl] + bufB[b, r, hh, sl]
              + pbuf[b, 2 * r + hh, sl]) * dv2
      return carry

    lax.fori_loop(0, PB, prow, 0)
    pltpu.sync_copy(pbuf.at[b], p1t_h.at[cid, pl.ds(2 * nb, 2 * PB)])

  for k in range(RPT // ZR):
    pltpu.sync_copy(zbuf, acc.at[pl.ds(sid * RPT + k * ZR, ZR)])
  pltpu.sync_copy(dst_h.at[wid], dstv)
  plsc.subcore_barrier()

  def no_scalar(g):
    del g

  for h in range(2):
    pltpu.sync_copy(srcs_h.at[h, wid], srcv)
    if h == 1:
      plsc.subcore_barrier()
    _prop_pipeline(p1t_h.at[cid], srcv, dstv, rows0, rows1, acc, sem2, sem3,
                   no_scalar)
    plsc.subcore_barrier()
    pltpu.sync_copy(acc.at[pl.ds(sid * RPT, RPT)],
                    parts_h.at[cid, pl.ds(sid * RPT, RPT), h])
    if h == 0:
      for k in range(RPT // ZR):
        pltpu.sync_copy(zbuf, acc.at[pl.ds(sid * RPT + k * ZR, ZR)])


_RB = 1024


def _final_body(parts_ref, p1_ref, dinv_ref, sparts_ref,
                w1_ref, b1_ref, w2_ref, b2_ref, out_ref):
  d = dinv_ref[0]
  p1r = p1_ref[0].reshape(_RB, 2, DH)
  q2 = (parts_ref[0] + parts_ref[1] + p1r) * d[:, None, None]
  w12 = jnp.dot(w1_ref[...], w2_ref[...], preferred_element_type=jnp.float32)
  b12 = jnp.dot(b1_ref[...][None, :], w2_ref[...],
                preferred_element_type=jnp.float32)[0]
  s = d * (jnp.sum(sparts_ref[...], axis=0) + d)
  out_ref[...] = (
      jnp.dot(q2[:, 0, :], w12[:DH, :], preferred_element_type=jnp.float32)
      + jnp.dot(q2[:, 1, :], w12[DH:, :], preferred_element_type=jnp.float32)
      + s[:, None] * b12[None, :] + b2_ref[...][None, :])


def _final_call(parts1, p1t, dinv, sparts, W1, b1, W2, b2):
  return pl.pallas_call(
      _final_body,
      grid=(NPAD // _RB,),
      in_specs=[
          pl.BlockSpec((NC, _RB, 2, DH), lambda i: (0, i, 0, 0)),
          pl.BlockSpec((1, 2 * _RB, DH), lambda i: (0, i, 0)),
          pl.BlockSpec((NC, _RB), lambda i: (0, i)),
          pl.BlockSpec((NW, _RB), lambda i: (0, i)),
          pl.BlockSpec((D, 2 * D), lambda i: (0, 0)),
          pl.BlockSpec((2 * D,), lambda i: (0,)),
          pl.BlockSpec((2 * D, D), lambda i: (0, 0)),
          pl.BlockSpec((D,), lambda i: (0,)),
      ],
      out_specs=pl.BlockSpec((_RB, D), lambda i: (i, 0)),
      out_shape=jax.ShapeDtypeStruct((NPAD, D), jnp.float32),
  )(parts1, p1t, dinv, sparts, W1, b1, W2, b2)


def kernel(edge_index, edge_weight, emb, W1, b1, W2, b2):
  src = edge_index[0].astype(jnp.int32)
  dst = edge_index[1].astype(jnp.int32)
  src2 = src * 2
  srcs = jnp.stack([src2, src2 + 1]).reshape(2, NW, NG, G)
  dst3 = dst.reshape(NW, NG, G)
  emb2 = jnp.pad(emb, ((0, NPAD - N_NODES), (0, 0))).reshape(2 * NPAD, DH)

  dinv = _pass1a(dst3)
  parts0, sparts, p0t = _pass1b(srcs, dst3, emb2, dinv)
  parts1, p1t = _pass2(srcs, dst3, dinv, parts0, p0t)
  out = _final_call(parts1, p1t, dinv, sparts, W1, b1, W2, b2)
  return out[:N_NODES]
